# baseline (speedup 1.0000x reference)
import functools

import jax
import jax.numpy as jnp
from jax import lax
from jax.experimental import pallas as pl
from jax.experimental.pallas import tpu as pltpu
from jax.experimental.pallas import tpu_sc as plsc

_NUM_CORES = 2
_NUM_SUBCORES = 16
_NW = _NUM_CORES * _NUM_SUBCORES
_L = 16

_B0, _B1 = 32, 69
_SH, _SW = 45, 90
_TH, _TW = 90, 180
_TGT = _TH * _TW
_TGT_PAD = _TGT + 8
_OCT = 8
_NUNIT = _B1 * (_B0 // _OCT)
_UPW = -(-_NUNIT // _NW)
_TCH = 10
_NCHUNK = _TH // _TCH
_SENT = _SH * 128
_MHALF = 8112


@jax.jit
def _interp(vt, idx_pad, mask_pad):
    mesh = plsc.VectorSubcoreMesh(
        core_axis_name="c",
        subcore_axis_name="s",
        num_cores=_NUM_CORES,
        num_subcores=_NUM_SUBCORES,
    )

    @functools.partial(
        pl.kernel,
        out_type=jax.ShapeDtypeStruct((_B1, _TH, _B0, _TW), jnp.float32),
        mesh=mesh,
        compiler_params=pltpu.CompilerParams(needs_layout_passes=False),
        scratch_types=[
            pltpu.VMEM((_TGT_PAD,), jnp.int32),
            pltpu.VMEM((_MHALF,), jnp.int32),
            pltpu.VMEM((_SH + 1, _OCT, _SW), jnp.float32),
            pltpu.VMEM((_TCH, _OCT, _TW), jnp.float32),
            pltpu.VMEM((_TCH, _OCT, _TW), jnp.float32),
            pltpu.SemaphoreType.DMA,
            pltpu.SemaphoreType.DMA,
        ],
    )
    def body(vt_hbm, idx_hbm, mask_hbm, out_hbm,
             idx_v, mask_v, tbl_v, st0_v, st1_v, sout0, sout1):
        wid = lax.axis_index("s") * _NUM_CORES + lax.axis_index("c")
        stage = (st0_v, st1_v)
        sout = (sout0, sout1)

        pltpu.sync_copy(idx_hbm, idx_v)
        sent_vec = jnp.full((_L,), _SENT, jnp.int32)

        for half, (hbase, hlen) in enumerate(((0, _MHALF), (_MHALF, _TGT_PAD - _MHALF))):
            pltpu.sync_copy(mask_hbm.at[pl.ds(hbase, hlen)],
                            mask_v.at[pl.ds(0, hlen)])

            @plsc.parallel_loop(0, hlen // _L, unroll=4)
            def _(g):
                sl = pl.ds(hbase + g * _L, _L)
                s = idx_v[sl]
                m = mask_v[pl.ds(g * _L, _L)]
                sh = (s * 46604) >> 22
                hw = (sh << 7) + (s - sh * 90)
                idx_v[sl] = jnp.where(m != 0, hw, sent_vec)

        nan_vec = jnp.full((_L,), jnp.nan, jnp.float32)
        for b in range(_OCT):
            tbl_v[_SH, b, pl.ds(0, _L)] = nan_vec

        def out_dst(b1, q, c):
            return out_hbm.at[b1, pl.ds(c * _TCH, _TCH), pl.ds(q * _OCT, _OCT), :]

        def fire(b1, q, c, buf):
            pltpu.async_copy(stage[buf], out_dst(b1, q, c), sout[buf])

        def absorb(buf):
            pltpu.make_async_copy(stage[buf], out_dst(0, 0, 0), sout[buf]).wait()

        def gather_chunk(c, buf):
            sbuf = stage[buf]

            @plsc.parallel_loop(0, _TCH)
            def _(thl):
                rowbase = (c * _TCH + thl) * _TW
                for wi in range(12):
                    w0 = wi * _L if wi < 11 else _TW - _L
                    hw = idx_v[pl.ds(rowbase + w0, _L)]
                    h = hw >> 7
                    w = hw & 127
                    for b0l in range(_OCT):
                        b0v = jnp.full((_L,), b0l, jnp.int32)
                        sbuf[thl, b0l, pl.ds(w0, _L)] = plsc.load_gather(
                            tbl_v, [h, b0v, w])

        def do_unit(i, u):
            b1 = u // 4
            q = u % 4
            pltpu.sync_copy(vt_hbm.at[b1, :, pl.ds(q * _OCT, _OCT), :],
                            tbl_v.at[pl.ds(0, _SH)])

            @pl.when(i > 0)
            def _():
                absorb(0)
            gather_chunk(0, 0)
            fire(b1, q, 0, 0)

            @pl.loop(0, (_NCHUNK - 1) // 2)
            def _(p):
                c1 = 2 * p + 1

                @pl.when((i > 0) | (p > 0))
                def _():
                    absorb(1)
                gather_chunk(c1, 1)
                fire(b1, q, c1, 1)

                absorb(0)
                gather_chunk(c1 + 1, 0)
                fire(b1, q, c1 + 1, 0)

        @pl.loop(0, _UPW)
        def _(i):
            u = wid + i * _NW

            @pl.when(u < _NUNIT)
            def _():
                do_unit(i, u)

        absorb(0)
        absorb(1)

    return body(vt, idx_pad, mask_pad)


def kernel(values, source_flat_index, valid_mask):
    vt = jnp.transpose(values, (1, 2, 0, 3))
    idx = jnp.pad(source_flat_index.astype(jnp.int32), (0, _TGT_PAD - _TGT))
    mask = jnp.pad(valid_mask.astype(jnp.int32), (0, _TGT_PAD - _TGT))
    out_t = _interp(vt, idx, mask)
    return jnp.transpose(out_t, (2, 0, 1, 3))


# speedup vs baseline: 1.4574x; 1.4574x over previous
import functools

import jax
import jax.numpy as jnp
from jax import lax
from jax.experimental import pallas as pl
from jax.experimental.pallas import tpu as pltpu
from jax.experimental.pallas import tpu_sc as plsc

_NUM_CORES = 2
_NUM_SUBCORES = 16
_NW = _NUM_CORES * _NUM_SUBCORES
_L = 16

_B0, _B1 = 32, 69
_SH, _SW = 45, 90
_TH, _TW = 90, 180
_TGT = _TH * _TW
_TGT_PAD = _TGT + 8
_OCT = 8
_NUNIT = _B1 * (_B0 // _OCT)
_UPW = -(-_NUNIT // _NW)
_TCH = 3
_NCHUNK = _TH // _TCH
_SENT = _SH * 128
_MCH = 2048


@jax.jit
def _interp(vt, idx_pad, mask_pad):
    mesh = plsc.VectorSubcoreMesh(
        core_axis_name="c",
        subcore_axis_name="s",
        num_cores=_NUM_CORES,
        num_subcores=_NUM_SUBCORES,
    )

    @functools.partial(
        pl.kernel,
        out_type=jax.ShapeDtypeStruct((_B1, _TH, _B0, _TW), jnp.float32),
        mesh=mesh,
        compiler_params=pltpu.CompilerParams(needs_layout_passes=False),
        scratch_types=[
            pltpu.VMEM((_TGT_PAD,), jnp.int32),
            pltpu.VMEM((_MCH,), jnp.int32),
            pltpu.VMEM((_SH + 1, _OCT, _SW), jnp.float32),
            pltpu.VMEM((_SH + 1, _OCT, _SW), jnp.float32),
            pltpu.VMEM((_TCH, _OCT, _TW), jnp.float32),
            pltpu.VMEM((_TCH, _OCT, _TW), jnp.float32),
            pltpu.SemaphoreType.DMA,
            pltpu.SemaphoreType.DMA,
            pltpu.SemaphoreType.DMA,
            pltpu.SemaphoreType.DMA,
        ],
    )
    def body(vt_hbm, idx_hbm, mask_hbm, out_hbm,
             idx_v, mask_v, tb0_v, tb1_v, st0_v, st1_v,
             stin0, stin1, sout0, sout1):
        wid = lax.axis_index("s") * _NUM_CORES + lax.axis_index("c")
        tblb = (tb0_v, tb1_v)
        stage = (st0_v, st1_v)
        stin = (stin0, stin1)
        sout = (sout0, sout1)

        def fire_tbl(u, tb):
            b1 = u // 4
            q = u % 4
            pltpu.async_copy(vt_hbm.at[b1, :, pl.ds(q * _OCT, _OCT), :],
                             tblb[tb].at[pl.ds(0, _SH)], stin[tb])

        def wait_tbl(tb):
            pltpu.make_async_copy(vt_hbm.at[0, :, pl.ds(0, _OCT), :],
                                  tblb[tb].at[pl.ds(0, _SH)], stin[tb]).wait()

        fire_tbl(wid, 0)

        pltpu.sync_copy(idx_hbm, idx_v)
        sent_vec = jnp.full((_L,), _SENT, jnp.int32)
        _chunks = [(k * _MCH, _MCH) for k in range(7)] + [(7 * _MCH, _TGT_PAD - 7 * _MCH)]
        for hbase, hlen in _chunks:
            pltpu.sync_copy(mask_hbm.at[pl.ds(hbase, hlen)],
                            mask_v.at[pl.ds(0, hlen)])

            @plsc.parallel_loop(0, hlen // _L, unroll=4)
            def _(g):
                sl = pl.ds(hbase + g * _L, _L)
                s = idx_v[sl]
                m = mask_v[pl.ds(g * _L, _L)]
                sh = (s * 46604) >> 22
                hw = (sh << 7) + (s - sh * 90)
                idx_v[sl] = jnp.where(m != 0, hw, sent_vec)

        nan_vec = jnp.full((_L,), jnp.nan, jnp.float32)
        for t in tblb:
            for b in range(_OCT):
                t[_SH, b, pl.ds(0, _L)] = nan_vec

        def out_dst(b1, q, c):
            return out_hbm.at[b1, pl.ds(c * _TCH, _TCH), pl.ds(q * _OCT, _OCT), :]

        def fire_out(b1, q, c, buf):
            pltpu.async_copy(stage[buf], out_dst(b1, q, c), sout[buf])

        def absorb(buf):
            pltpu.make_async_copy(stage[buf], out_dst(0, 0, 0), sout[buf]).wait()

        def gather_chunk(c, tb, buf):
            sbuf = stage[buf]
            tbl_v = tblb[tb]

            @plsc.parallel_loop(0, _TCH * _OCT, unroll=2)
            def _(l):
                thl = l >> 3
                b0l = l & 7
                b0v = jnp.zeros((_L,), jnp.int32) + b0l
                rowbase = (c * _TCH + thl) * _TW
                for wi in range(12):
                    w0 = wi * _L if wi < 11 else _TW - _L
                    hw = idx_v[pl.ds(rowbase + w0, _L)]
                    h = hw >> 7
                    w = hw & 127
                    sbuf[thl, b0l, pl.ds(w0, _L)] = plsc.load_gather(
                        tbl_v, [h, b0v, w])

        def chunks(u, tb, first):
            b1 = u // 4
            q = u % 4
            for c, buf in ((0, 0), (1, 1)):
                if not first:
                    absorb(buf)
                gather_chunk(c, tb, buf)
                fire_out(b1, q, c, buf)

            @pl.loop(0, (_NCHUNK - 2) // 2)
            def _(p):
                for off, buf in ((2, 0), (3, 1)):
                    c = 2 * p + off
                    absorb(buf)
                    gather_chunk(c, tb, buf)
                    fire_out(b1, q, c, buf)

        wait_tbl(0)
        fire_tbl(wid + _NW, 1)
        chunks(wid, 0, True)

        @pl.loop(0, (_UPW - 1) // 2)
        def _(p):
            u1 = wid + (2 * p + 1) * _NW
            u2 = u1 + _NW
            u3 = u2 + _NW

            wait_tbl(1)

            @pl.when(u2 < _NUNIT)
            def _():
                fire_tbl(u2, 0)
            chunks(u1, 1, False)

            @pl.when(u2 < _NUNIT)
            def _():
                wait_tbl(0)

                @pl.when(u3 < _NUNIT)
                def _():
                    fire_tbl(u3, 1)
                chunks(u2, 0, False)

        absorb(0)
        absorb(1)

    return body(vt, idx_pad, mask_pad)


def kernel(values, source_flat_index, valid_mask):
    vt = jnp.transpose(values, (1, 2, 0, 3))
    idx = jnp.pad(source_flat_index.astype(jnp.int32), (0, _TGT_PAD - _TGT))
    mask = jnp.pad(valid_mask.astype(jnp.int32), (0, _TGT_PAD - _TGT))
    out_t = _interp(vt, idx, mask)
    return jnp.transpose(out_t, (2, 0, 1, 3))


# speedup vs baseline: 1.4850x; 1.0189x over previous
import functools

import jax
import jax.numpy as jnp
from jax import lax
from jax.experimental import pallas as pl
from jax.experimental.pallas import tpu as pltpu
from jax.experimental.pallas import tpu_sc as plsc

_NUM_CORES = 2
_NUM_SUBCORES = 16
_NW = _NUM_CORES * _NUM_SUBCORES
_L = 16

_B0, _B1 = 32, 69
_SH, _SW = 45, 90
_TH, _TW = 90, 180
_TGT = _TH * _TW
_TGT_PAD = _TGT + 8
_OCT = 8
_NUNIT = _B1 * (_B0 // _OCT)
_UPW = -(-_NUNIT // _NW)
_TCH = 9
_NCHUNK = _TH // _TCH
_SENT = _SH * 128
_MCH = 2048


@jax.jit
def _interp(vt, idx_pad, mask_pad):
    mesh = plsc.VectorSubcoreMesh(
        core_axis_name="c",
        subcore_axis_name="s",
        num_cores=_NUM_CORES,
        num_subcores=_NUM_SUBCORES,
    )

    @functools.partial(
        pl.kernel,
        out_type=jax.ShapeDtypeStruct((_B1, _TH, _B0, _TW), jnp.float32),
        mesh=mesh,
        compiler_params=pltpu.CompilerParams(needs_layout_passes=False),
        scratch_types=[
            pltpu.VMEM((_TGT_PAD,), jnp.int32),
            pltpu.VMEM((_MCH,), jnp.int32),
            pltpu.VMEM((_SH + 1, _OCT, _SW), jnp.float32),
            pltpu.VMEM((_TCH, _OCT, _TW), jnp.float32),
            pltpu.VMEM((_TCH, _OCT, _TW), jnp.float32),
            pltpu.SemaphoreType.DMA,
            pltpu.SemaphoreType.DMA,
            pltpu.SemaphoreType.DMA,
        ],
    )
    def body(vt_hbm, idx_hbm, mask_hbm, out_hbm,
             idx_v, mask_v, tbl_v, st0_v, st1_v,
             stin, sout0, sout1):
        wid = lax.axis_index("s") * _NUM_CORES + lax.axis_index("c")
        stage = (st0_v, st1_v)
        sout = (sout0, sout1)

        def tbl_src(u):
            b1 = u // 4
            q = u % 4
            return vt_hbm.at[b1, :, pl.ds(q * _OCT, _OCT), :]

        pltpu.async_copy(tbl_src(wid), tbl_v.at[pl.ds(0, _SH)], stin)

        pltpu.sync_copy(idx_hbm, idx_v)
        sent_vec = jnp.full((_L,), _SENT, jnp.int32)
        _chunks = [(k * _MCH, _MCH) for k in range(7)] + [(7 * _MCH, _TGT_PAD - 7 * _MCH)]
        for hbase, hlen in _chunks:
            pltpu.sync_copy(mask_hbm.at[pl.ds(hbase, hlen)],
                            mask_v.at[pl.ds(0, hlen)])

            @plsc.parallel_loop(0, hlen // _L, unroll=4)
            def _(g):
                sl = pl.ds(hbase + g * _L, _L)
                s = idx_v[sl]
                m = mask_v[pl.ds(g * _L, _L)]
                sh = (s * 46604) >> 22
                hw = (sh << 7) + (s - sh * 90)
                idx_v[sl] = jnp.where(m != 0, hw, sent_vec)

        nan_vec = jnp.full((_L,), jnp.nan, jnp.float32)
        for b in range(_OCT):
            tbl_v[_SH, b, pl.ds(0, _L)] = nan_vec

        def out_dst(b1, q, c):
            return out_hbm.at[b1, pl.ds(c * _TCH, _TCH), pl.ds(q * _OCT, _OCT), :]

        def fire_out(b1, q, c, buf):
            pltpu.async_copy(stage[buf], out_dst(b1, q, c), sout[buf])

        def absorb(buf):
            pltpu.make_async_copy(stage[buf], out_dst(0, 0, 0), sout[buf]).wait()

        def gather_chunk(c, buf):
            sbuf = stage[buf]

            @plsc.parallel_loop(0, _TCH * _OCT, unroll=2)
            def _(l):
                thl = l >> 3
                b0l = l & 7
                b0v = jnp.zeros((_L,), jnp.int32) + b0l
                rowbase = (c * _TCH + thl) * _TW
                for wi in range(12):
                    w0 = wi * _L if wi < 11 else _TW - _L
                    hw = idx_v[pl.ds(rowbase + w0, _L)]
                    h = hw >> 7
                    w = hw & 127
                    sbuf[thl, b0l, pl.ds(w0, _L)] = plsc.load_gather(
                        tbl_v, [h, b0v, w])

        def chunks(u, first):
            b1 = u // 4
            q = u % 4
            for c, buf in ((0, 0), (1, 1)):
                if not first:
                    absorb(buf)
                gather_chunk(c, buf)
                fire_out(b1, q, c, buf)

            @pl.loop(0, (_NCHUNK - 2) // 2)
            def _(p):
                for off, buf in ((2, 0), (3, 1)):
                    c = 2 * p + off
                    absorb(buf)
                    gather_chunk(c, buf)
                    fire_out(b1, q, c, buf)

        pltpu.make_async_copy(tbl_src(wid), tbl_v.at[pl.ds(0, _SH)], stin).wait()
        chunks(wid, True)

        @pl.loop(1, _UPW)
        def _(i):
            u = wid + i * _NW

            @pl.when(u < _NUNIT)
            def _():
                pltpu.sync_copy(tbl_src(u), tbl_v.at[pl.ds(0, _SH)])
                chunks(u, False)

        absorb(0)
        absorb(1)

    return body(vt, idx_pad, mask_pad)


def kernel(values, source_flat_index, valid_mask):
    vt = jnp.transpose(values, (1, 2, 0, 3))
    idx = jnp.pad(source_flat_index.astype(jnp.int32), (0, _TGT_PAD - _TGT))
    mask = jnp.pad(valid_mask.astype(jnp.int32), (0, _TGT_PAD - _TGT))
    out_t = _interp(vt, idx, mask)
    return jnp.transpose(out_t, (2, 0, 1, 3))


# speedup vs baseline: 1.5432x; 1.0392x over previous
import functools

import jax
import jax.numpy as jnp
from jax import lax
from jax.experimental import pallas as pl
from jax.experimental.pallas import tpu as pltpu
from jax.experimental.pallas import tpu_sc as plsc

_NUM_CORES = 2
_NUM_SUBCORES = 16
_NW = _NUM_CORES * _NUM_SUBCORES
_L = 16

_B0, _B1 = 32, 69
_SH, _SW = 45, 90
_TH, _TW = 90, 180
_TGT = _TH * _TW
_TGT_PAD = _TGT + 8
_OCT = 8
_NUNIT = _B1 * (_B0 // _OCT)
_UPW = -(-_NUNIT // _NW)
_TCH = 10
_NCHUNK = _TH // _TCH
_SENT = _SH * 128
_MHALF = 8112


@jax.jit
def _interp(vt, idx_pad, mask_pad):
    mesh = plsc.VectorSubcoreMesh(
        core_axis_name="c",
        subcore_axis_name="s",
        num_cores=_NUM_CORES,
        num_subcores=_NUM_SUBCORES,
    )

    @functools.partial(
        pl.kernel,
        out_type=jax.ShapeDtypeStruct((_B1, _TH, _B0, _TW), jnp.float32),
        mesh=mesh,
        compiler_params=pltpu.CompilerParams(needs_layout_passes=False),
        scratch_types=[
            pltpu.VMEM((_TGT_PAD,), jnp.int32),
            pltpu.VMEM((_MHALF,), jnp.int32),
            pltpu.VMEM((_SH + 1, _OCT, _SW), jnp.float32),
            pltpu.VMEM((_TCH, _OCT, _TW), jnp.float32),
            pltpu.VMEM((_TCH, _OCT, _TW), jnp.float32),
            pltpu.SemaphoreType.DMA,
            pltpu.SemaphoreType.DMA,
            pltpu.SemaphoreType.DMA,
        ],
    )
    def body(vt_hbm, idx_hbm, mask_hbm, out_hbm,
             idx_v, mask_v, tbl_v, st0_v, st1_v,
             stin, sout0, sout1):
        wid = lax.axis_index("s") * _NUM_CORES + lax.axis_index("c")
        stage = (st0_v, st1_v)
        sout = (sout0, sout1)

        def tbl_src(u):
            b1 = u // 4
            q = u % 4
            return vt_hbm.at[b1, :, pl.ds(q * _OCT, _OCT), :]

        pltpu.async_copy(tbl_src(wid), tbl_v.at[pl.ds(0, _SH)], stin)

        pltpu.sync_copy(idx_hbm, idx_v)
        sent_vec = jnp.full((_L,), _SENT, jnp.int32)
        for hbase, hlen in ((0, _MHALF), (_MHALF, _TGT_PAD - _MHALF)):
            pltpu.sync_copy(mask_hbm.at[pl.ds(hbase, hlen)],
                            mask_v.at[pl.ds(0, hlen)])

            @plsc.parallel_loop(0, hlen // _L, unroll=4)
            def _(g):
                sl = pl.ds(hbase + g * _L, _L)
                s = idx_v[sl]
                m = mask_v[pl.ds(g * _L, _L)]
                sh = (s * 46604) >> 22
                hw = (sh << 7) + (s - sh * 90)
                idx_v[sl] = jnp.where(m != 0, hw, sent_vec)

        nan_vec = jnp.full((_L,), jnp.nan, jnp.float32)
        for b in range(_OCT):
            tbl_v[_SH, b, pl.ds(0, _L)] = nan_vec

        def out_dst(b1, q, c):
            return out_hbm.at[b1, pl.ds(c * _TCH, _TCH), pl.ds(q * _OCT, _OCT), :]

        def fire_out(b1, q, c, buf):
            pltpu.async_copy(stage[buf], out_dst(b1, q, c), sout[buf])

        def absorb(buf):
            pltpu.make_async_copy(stage[buf], out_dst(0, 0, 0), sout[buf]).wait()

        def gather_chunk(c, buf):
            sbuf = stage[buf]

            @plsc.parallel_loop(0, _TCH * _OCT, unroll=2)
            def _(l):
                thl = l >> 3
                b0l = l & 7
                b0v = jnp.zeros((_L,), jnp.int32) + b0l
                rowbase = (c * _TCH + thl) * _TW
                for wi in range(12):
                    w0 = wi * _L if wi < 11 else _TW - _L
                    hw = idx_v[pl.ds(rowbase + w0, _L)]
                    h = hw >> 7
                    w = hw & 127
                    sbuf[thl, b0l, pl.ds(w0, _L)] = plsc.load_gather(
                        tbl_v, [h, b0v, w])

        def chunks(u, first):
            b1 = u // 4
            q = u % 4
            if not first:
                absorb(0)
            gather_chunk(0, 0)
            fire_out(b1, q, 0, 0)

            @pl.loop(0, (_NCHUNK - 1) // 2)
            def _(p):
                c1 = 2 * p + 1
                if first:
                    @pl.when(p > 0)
                    def _():
                        absorb(1)
                else:
                    absorb(1)
                gather_chunk(c1, 1)
                fire_out(b1, q, c1, 1)

                absorb(0)
                gather_chunk(c1 + 1, 0)
                fire_out(b1, q, c1 + 1, 0)

        pltpu.make_async_copy(tbl_src(wid), tbl_v.at[pl.ds(0, _SH)], stin).wait()
        chunks(wid, True)

        @pl.loop(1, _UPW)
        def _(i):
            u = wid + i * _NW

            @pl.when(u < _NUNIT)
            def _():
                pltpu.sync_copy(tbl_src(u), tbl_v.at[pl.ds(0, _SH)])
                chunks(u, False)

        absorb(0)
        absorb(1)

    return body(vt, idx_pad, mask_pad)


def kernel(values, source_flat_index, valid_mask):
    vt = jnp.transpose(values, (1, 2, 0, 3))
    idx = jnp.pad(source_flat_index.astype(jnp.int32), (0, _TGT_PAD - _TGT))
    mask = jnp.pad(valid_mask.astype(jnp.int32), (0, _TGT_PAD - _TGT))
    out_t = _interp(vt, idx, mask)
    return jnp.transpose(out_t, (2, 0, 1, 3))


# speedup vs baseline: 1.5589x; 1.0102x over previous
import functools

import jax
import jax.numpy as jnp
from jax import lax
from jax.experimental import pallas as pl
from jax.experimental.pallas import tpu as pltpu
from jax.experimental.pallas import tpu_sc as plsc

_NUM_CORES = 2
_NUM_SUBCORES = 16
_NW = _NUM_CORES * _NUM_SUBCORES
_L = 16

_B0, _B1 = 32, 69
_SH, _SW = 45, 90
_TH, _TW = 90, 180
_TGT = _TH * _TW
_TGT_PAD = _TGT + 8
_OCT = 8
_NUNIT = _B1 * (_B0 // _OCT)
_UPW = -(-_NUNIT // _NW)
_TCH = 10
_NCHUNK = _TH // _TCH
_SENT = _SH * 128
_MHALF = 8112


@jax.jit
def _interp(vt, idx_pad, mask_pad):
    mesh = plsc.VectorSubcoreMesh(
        core_axis_name="c",
        subcore_axis_name="s",
        num_cores=_NUM_CORES,
        num_subcores=_NUM_SUBCORES,
    )

    @functools.partial(
        pl.kernel,
        out_type=jax.ShapeDtypeStruct((_B1, _TH, _B0, _TW), jnp.float32),
        mesh=mesh,
        compiler_params=pltpu.CompilerParams(needs_layout_passes=False),
        scratch_types=[
            pltpu.VMEM((_TGT_PAD,), jnp.int32),
            pltpu.VMEM((_MHALF,), jnp.int32),
            pltpu.VMEM((_SH + 1, _OCT, _SW), jnp.float32),
            pltpu.VMEM((_TCH, _OCT, _TW), jnp.float32),
            pltpu.VMEM((_TCH, _OCT, _TW), jnp.float32),
            pltpu.SemaphoreType.DMA,
            pltpu.SemaphoreType.DMA,
            pltpu.SemaphoreType.DMA,
        ],
    )
    def body(vt_hbm, idx_hbm, mask_hbm, out_hbm,
             idx_v, mask_v, tbl_v, st0_v, st1_v,
             stin, sout0, sout1):
        wid = lax.axis_index("s") * _NUM_CORES + lax.axis_index("c")
        stage = (st0_v, st1_v)
        sout = (sout0, sout1)

        def tbl_src(u):
            b1 = u // 4
            q = u % 4
            return vt_hbm.at[b1, :, pl.ds(q * _OCT, _OCT), :]

        pltpu.async_copy(tbl_src(wid), tbl_v.at[pl.ds(0, _SH)], stin)

        pltpu.sync_copy(idx_hbm, idx_v)
        sent_vec = jnp.full((_L,), _SENT, jnp.int32)
        for hbase, hlen in ((0, _MHALF), (_MHALF, _TGT_PAD - _MHALF)):
            pltpu.sync_copy(mask_hbm.at[pl.ds(hbase, hlen)],
                            mask_v.at[pl.ds(0, hlen)])

            @plsc.parallel_loop(0, hlen // _L, unroll=4)
            def _(g):
                sl = pl.ds(hbase + g * _L, _L)
                s = idx_v[sl]
                m = mask_v[pl.ds(g * _L, _L)]
                sh = (s * 46604) >> 22
                hw = (sh << 7) + (s - sh * 90)
                idx_v[sl] = jnp.where(m != 0, hw, sent_vec)

        nan_vec = jnp.full((_L,), jnp.nan, jnp.float32)
        for b in range(_OCT):
            tbl_v[_SH, b, pl.ds(0, _L)] = nan_vec

        def out_dst(b1, q, c):
            return out_hbm.at[b1, pl.ds(c * _TCH, _TCH), pl.ds(q * _OCT, _OCT), :]

        def fire_out(b1, q, c, buf):
            pltpu.async_copy(stage[buf], out_dst(b1, q, c), sout[buf])

        def absorb(buf):
            pltpu.make_async_copy(stage[buf], out_dst(0, 0, 0), sout[buf]).wait()

        def gather_chunk(c, buf):
            sbuf = stage[buf]

            @plsc.parallel_loop(0, _TCH * _OCT // 2, unroll=2)
            def _(l):
                thl = l >> 2
                b0a = (l & 3) * 2
                va = jnp.zeros((_L,), jnp.int32) + b0a
                vb = va + 1
                rowbase = (c * _TCH + thl) * _TW
                for wi in range(12):
                    w0 = wi * _L if wi < 11 else _TW - _L
                    hw = idx_v[pl.ds(rowbase + w0, _L)]
                    h = hw >> 7
                    w = hw & 127
                    sbuf[thl, b0a, pl.ds(w0, _L)] = plsc.load_gather(
                        tbl_v, [h, va, w])
                    sbuf[thl, b0a + 1, pl.ds(w0, _L)] = plsc.load_gather(
                        tbl_v, [h, vb, w])

        def chunks(u, first):
            b1 = u // 4
            q = u % 4
            if not first:
                absorb(0)
            gather_chunk(0, 0)
            fire_out(b1, q, 0, 0)

            @pl.loop(0, (_NCHUNK - 1) // 2)
            def _(p):
                c1 = 2 * p + 1
                if first:
                    @pl.when(p > 0)
                    def _():
                        absorb(1)
                else:
                    absorb(1)
                gather_chunk(c1, 1)
                fire_out(b1, q, c1, 1)

                absorb(0)
                gather_chunk(c1 + 1, 0)
                fire_out(b1, q, c1 + 1, 0)

        pltpu.make_async_copy(tbl_src(wid), tbl_v.at[pl.ds(0, _SH)], stin).wait()
        chunks(wid, True)

        @pl.loop(1, _UPW)
        def _(i):
            u = wid + i * _NW

            @pl.when(u < _NUNIT)
            def _():
                pltpu.sync_copy(tbl_src(u), tbl_v.at[pl.ds(0, _SH)])
                chunks(u, False)

        absorb(0)
        absorb(1)

    return body(vt, idx_pad, mask_pad)


def kernel(values, source_flat_index, valid_mask):
    vt = jnp.transpose(values, (1, 2, 0, 3))
    idx = jnp.pad(source_flat_index.astype(jnp.int32), (0, _TGT_PAD - _TGT))
    mask = jnp.pad(valid_mask.astype(jnp.int32), (0, _TGT_PAD - _TGT))
    out_t = _interp(vt, idx, mask)
    return jnp.transpose(out_t, (2, 0, 1, 3))
